# baseline (speedup 1.0000x reference)
import functools

import jax
import jax.numpy as jnp
from jax import lax
from jax.experimental import pallas as pl
from jax.experimental.pallas import tpu as pltpu
from jax.experimental.pallas import tpu_sc as plsc

EMB = 64
ROW = 2 * EMB
LANES = 16
VPR = EMB // LANES

_info = plsc.get_sparse_core_info()
NC, NS = _info.num_cores, _info.num_subcores
NW = NC * NS


def _tc_fuse_body(wt_ref, wc_ref, o_ref):
    o_ref[...] = jnp.concatenate([wt_ref[...], wc_ref[...]], axis=1)


def _tc_fuse(wt, wc, V):
    C = 8000
    return pl.pallas_call(
        _tc_fuse_body,
        grid=(V // C,),
        in_specs=[
            pl.BlockSpec((C, EMB), lambda i: (i, 0)),
            pl.BlockSpec((C, EMB), lambda i: (i, 0)),
        ],
        out_specs=pl.BlockSpec((C, ROW), lambda i: (i, 0)),
        out_shape=jax.ShapeDtypeStruct((V, ROW), jnp.float32),
    )(wt, wc)


def _sc_partials(B, NEG, V):
    CHUNK = B // NW
    SB = 16
    STEPS = CHUNK // SB
    RPS = SB * NEG
    NROWS = CHUNK * NEG // 128
    assert CHUNK % SB == 0 and B % NW == 0 and RPS == 320

    mesh = plsc.VectorSubcoreMesh(core_axis_name="c", subcore_axis_name="s")

    @functools.partial(
        pl.kernel,
        mesh=mesh,
        compiler_params=pltpu.CompilerParams(use_tc_tiling_on_sc=False),
        out_type=[
            jax.ShapeDtypeStruct((B, LANES), jnp.float32),
            jax.ShapeDtypeStruct((B, LANES), jnp.float32),
        ],
        scratch_types=[
            pltpu.VMEM((STEPS, SB), jnp.int32),
            pltpu.VMEM((STEPS, SB), jnp.int32),
            pltpu.VMEM((NROWS, 128), jnp.int32),
            pltpu.VMEM((SB, ROW), jnp.float32),
            pltpu.VMEM((SB, ROW), jnp.float32),
            pltpu.VMEM((SB, ROW), jnp.float32),
            pltpu.VMEM((SB, ROW), jnp.float32),
            pltpu.VMEM((RPS, ROW), jnp.float32),
            pltpu.VMEM((RPS, ROW), jnp.float32),
            pltpu.VMEM((SB, LANES), jnp.float32),
            pltpu.VMEM((SB, LANES), jnp.float32),
            pltpu.VMEM((SB, LANES), jnp.float32),
            pltpu.VMEM((SB, LANES), jnp.float32),
            pltpu.SemaphoreType.DMA,
            pltpu.SemaphoreType.DMA,
            pltpu.SemaphoreType.DMA,
            pltpu.SemaphoreType.DMA,
        ],
    )
    def sc_kernel(tgt_idx_hbm, ctx_idx_hbm, neg_idx_hbm, w2_hbm,
                  pos_out, neg_out,
                  tgt_idx_v, ctx_idx_v, neg_idx_v,
                  tb0, tb1, cb0, cb1, rb0, rb1,
                  op0, op1, on0, on1, sem0, sem1, osem0, osem1):
        wid = lax.axis_index("s") * NC + lax.axis_index("c")
        tb = (tb0, tb1)
        cb = (cb0, cb1)
        rb = (rb0, rb1)
        op = (op0, op1)
        on = (on0, on1)
        sems = (sem0, sem1)
        osems = (osem0, osem1)

        pltpu.sync_copy(tgt_idx_hbm.at[pl.ds(wid * STEPS, STEPS), :], tgt_idx_v)
        pltpu.sync_copy(ctx_idx_hbm.at[pl.ds(wid * STEPS, STEPS), :], ctx_idx_v)
        pltpu.sync_copy(neg_idx_hbm.at[pl.ds(wid * NROWS, NROWS), :], neg_idx_v)

        def issue(s, p, even):
            j = s // 2
            if even:
                parts = ((5 * j, 0, 128, 0), (5 * j + 1, 0, 128, 128),
                         (5 * j + 2, 0, 64, 256))
            else:
                parts = ((5 * j + 2, 64, 64, 0), (5 * j + 3, 0, 128, 64),
                         (5 * j + 4, 0, 128, 192))
            for row, off, n, dst in parts:
                pltpu.async_copy(
                    w2_hbm.at[neg_idx_v.at[row, pl.ds(off, n)]],
                    rb[p].at[pl.ds(dst, n), :],
                    sems[p])
            pltpu.async_copy(w2_hbm.at[tgt_idx_v.at[s]], tb[p], sems[p])
            pltpu.async_copy(w2_hbm.at[ctx_idx_v.at[s]], cb[p], sems[p])

        def drain(p):
            pltpu.make_async_copy(w2_hbm.at[pl.ds(0, RPS), :], rb[p], sems[p]).wait()
            pltpu.make_async_copy(w2_hbm.at[pl.ds(0, SB), :], tb[p], sems[p]).wait()
            pltpu.make_async_copy(w2_hbm.at[pl.ds(0, SB), :], cb[p], sems[p]).wait()

        def odrain(p):
            pltpu.make_async_copy(pos_out.at[pl.ds(0, SB), :], op[p], osems[p]).wait()
            pltpu.make_async_copy(pos_out.at[pl.ds(0, SB), :], on[p], osems[p]).wait()

        def compute(s, p):
            rbp, tbp, cbp = rb[p], tb[p], cb[p]

            def bbody(b, carry):
                r0 = b * NEG
                accs = [rbp[r0, pl.ds(EMB + k * LANES, LANES)]
                        for k in range(VPR)]
                for j in range(1, NEG):
                    for k in range(VPR):
                        accs[k] = accs[k] + rbp[r0 + j,
                                                pl.ds(EMB + k * LANES, LANES)]
                npart = None
                ppart = None
                for k in range(VPR):
                    tk = tbp[b, pl.ds(k * LANES, LANES)]
                    ck = cbp[b, pl.ds(EMB + k * LANES, LANES)]
                    nk = accs[k] * tk
                    pk = tk * ck
                    npart = nk if npart is None else npart + nk
                    ppart = pk if ppart is None else ppart + pk
                op[p][b, :] = ppart
                on[p][b, :] = npart
                return carry

            lax.fori_loop(0, SB, bbody, 0)
            base = wid * CHUNK + s * SB
            pltpu.async_copy(op[p], pos_out.at[pl.ds(base, SB), :], osems[p])
            pltpu.async_copy(on[p], neg_out.at[pl.ds(base, SB), :], osems[p])

        issue(0, 0, True)
        issue(1, 1, False)
        drain(0)
        compute(0, 0)
        issue(2, 0, True)
        drain(1)
        compute(1, 1)
        issue(3, 1, False)

        def lbody(i, carry):
            for p in range(2):
                cur = 2 * i + 2 + p
                drain(p)
                odrain(p)
                compute(cur, p)
                issue(cur + 2, p, p == 0)
            return carry

        lax.fori_loop(0, STEPS // 2 - 2, lbody, 0)
        for p in range(2):
            drain(p)
            odrain(p)
            compute(STEPS - 2 + p, p)
        odrain(0)
        odrain(1)

    return sc_kernel


def _tc_reduce_body(p_ref, n_ref, o_ref):
    ri = lax.broadcasted_iota(jnp.int32, (128, 128 // LANES), 0)
    cj = lax.broadcasted_iota(jnp.int32, (128, 128 // LANES), 1)
    sel = jnp.where(ri // LANES == cj, 1.0, 0.0).astype(jnp.float32)
    pos = lax.dot(p_ref[...], sel, precision=lax.Precision.HIGHEST)
    neg = lax.dot(n_ref[...], sel, precision=lax.Precision.HIGHEST)

    def logsig(x):
        return jnp.minimum(x, 0.0) - jnp.log1p(jnp.exp(-jnp.abs(x)))

    o_ref[0, 0] = -(jnp.sum(logsig(pos)) + jnp.sum(logsig(-neg)))


def kernel(target_word, context_word, negative_example, W_target, W_context):
    B = target_word.shape[0]
    NEG = negative_example.shape[1]
    V = W_target.shape[0]

    t2 = target_word.astype(jnp.int32).reshape(-1, 16)
    c2 = context_word.astype(jnp.int32).reshape(-1, 16)
    n2 = negative_example.astype(jnp.int32).reshape(B * NEG // 128, 128)

    w2 = _tc_fuse(W_target, W_context, V)
    pos_p, neg_p = _sc_partials(B, NEG, V)(t2, c2, n2, w2)

    reduce_call = pl.pallas_call(
        _tc_reduce_body,
        out_shape=jax.ShapeDtypeStruct((1, 1), jnp.float32),
        out_specs=pl.BlockSpec(memory_space=pltpu.SMEM),
    )
    loss = reduce_call(pos_p.reshape(-1, 128), neg_p.reshape(-1, 128))
    return loss[0, 0]


# speedup vs baseline: 1.0001x; 1.0001x over previous
import functools

import jax
import jax.numpy as jnp
from jax import lax
from jax.experimental import pallas as pl
from jax.experimental.pallas import tpu as pltpu
from jax.experimental.pallas import tpu_sc as plsc

EMB = 64
ROW = 2 * EMB
LANES = 16
VPR = EMB // LANES

_info = plsc.get_sparse_core_info()
NC, NS = _info.num_cores, _info.num_subcores
NW = NC * NS


def _tc_fuse_body(wt_ref, wc_ref, o_ref):
    o_ref[:, :EMB] = wt_ref[...]
    o_ref[:, EMB:] = wc_ref[...]


def _tc_fuse(wt, wc, V):
    C = 8000
    return pl.pallas_call(
        _tc_fuse_body,
        grid=(V // C,),
        in_specs=[
            pl.BlockSpec((C, EMB), lambda i: (i, 0)),
            pl.BlockSpec((C, EMB), lambda i: (i, 0)),
        ],
        out_specs=pl.BlockSpec((C, ROW), lambda i: (i, 0)),
        out_shape=jax.ShapeDtypeStruct((V, ROW), jnp.float32),
    )(wt, wc)


def _sc_partials(B, NEG, V):
    CHUNK = B // NW
    SB = 16
    STEPS = CHUNK // SB
    RPS = SB * NEG
    NROWS = CHUNK * NEG // 128
    assert CHUNK % SB == 0 and B % NW == 0 and RPS == 320

    mesh = plsc.VectorSubcoreMesh(core_axis_name="c", subcore_axis_name="s")

    @functools.partial(
        pl.kernel,
        mesh=mesh,
        compiler_params=pltpu.CompilerParams(use_tc_tiling_on_sc=False),
        out_type=[
            jax.ShapeDtypeStruct((B, LANES), jnp.float32),
            jax.ShapeDtypeStruct((B, LANES), jnp.float32),
        ],
        scratch_types=[
            pltpu.VMEM((STEPS, SB), jnp.int32),
            pltpu.VMEM((STEPS, SB), jnp.int32),
            pltpu.VMEM((NROWS, 128), jnp.int32),
            pltpu.VMEM((SB, ROW), jnp.float32),
            pltpu.VMEM((SB, ROW), jnp.float32),
            pltpu.VMEM((SB, ROW), jnp.float32),
            pltpu.VMEM((SB, ROW), jnp.float32),
            pltpu.VMEM((RPS, ROW), jnp.float32),
            pltpu.VMEM((RPS, ROW), jnp.float32),
            pltpu.VMEM((SB, LANES), jnp.float32),
            pltpu.VMEM((SB, LANES), jnp.float32),
            pltpu.VMEM((SB, LANES), jnp.float32),
            pltpu.VMEM((SB, LANES), jnp.float32),
            pltpu.SemaphoreType.DMA,
            pltpu.SemaphoreType.DMA,
            pltpu.SemaphoreType.DMA,
            pltpu.SemaphoreType.DMA,
        ],
    )
    def sc_kernel(tgt_idx_hbm, ctx_idx_hbm, neg_idx_hbm, w2_hbm,
                  pos_out, neg_out,
                  tgt_idx_v, ctx_idx_v, neg_idx_v,
                  tb0, tb1, cb0, cb1, rb0, rb1,
                  op0, op1, on0, on1, sem0, sem1, osem0, osem1):
        wid = lax.axis_index("s") * NC + lax.axis_index("c")
        tb = (tb0, tb1)
        cb = (cb0, cb1)
        rb = (rb0, rb1)
        op = (op0, op1)
        on = (on0, on1)
        sems = (sem0, sem1)
        osems = (osem0, osem1)

        pltpu.sync_copy(tgt_idx_hbm.at[pl.ds(wid * STEPS, STEPS), :], tgt_idx_v)
        pltpu.sync_copy(ctx_idx_hbm.at[pl.ds(wid * STEPS, STEPS), :], ctx_idx_v)
        pltpu.sync_copy(neg_idx_hbm.at[pl.ds(wid * NROWS, NROWS), :], neg_idx_v)

        def issue(s, p, even):
            j = s // 2
            if even:
                parts = ((5 * j, 0, 128, 0), (5 * j + 1, 0, 128, 128),
                         (5 * j + 2, 0, 64, 256))
            else:
                parts = ((5 * j + 2, 64, 64, 0), (5 * j + 3, 0, 128, 64),
                         (5 * j + 4, 0, 128, 192))
            for row, off, n, dst in parts:
                pltpu.async_copy(
                    w2_hbm.at[neg_idx_v.at[row, pl.ds(off, n)]],
                    rb[p].at[pl.ds(dst, n), :],
                    sems[p])
            pltpu.async_copy(w2_hbm.at[tgt_idx_v.at[s]], tb[p], sems[p])
            pltpu.async_copy(w2_hbm.at[ctx_idx_v.at[s]], cb[p], sems[p])

        def drain(p):
            pltpu.make_async_copy(w2_hbm.at[pl.ds(0, RPS), :], rb[p], sems[p]).wait()
            pltpu.make_async_copy(w2_hbm.at[pl.ds(0, SB), :], tb[p], sems[p]).wait()
            pltpu.make_async_copy(w2_hbm.at[pl.ds(0, SB), :], cb[p], sems[p]).wait()

        def odrain(p):
            pltpu.make_async_copy(pos_out.at[pl.ds(0, SB), :], op[p], osems[p]).wait()
            pltpu.make_async_copy(pos_out.at[pl.ds(0, SB), :], on[p], osems[p]).wait()

        def compute(s, p):
            rbp, tbp, cbp = rb[p], tb[p], cb[p]

            def bbody(b, carry):
                r0 = b * NEG
                accs = [rbp[r0, pl.ds(EMB + k * LANES, LANES)]
                        for k in range(VPR)]
                for j in range(1, NEG):
                    for k in range(VPR):
                        accs[k] = accs[k] + rbp[r0 + j,
                                                pl.ds(EMB + k * LANES, LANES)]
                npart = None
                ppart = None
                for k in range(VPR):
                    tk = tbp[b, pl.ds(k * LANES, LANES)]
                    ck = cbp[b, pl.ds(EMB + k * LANES, LANES)]
                    nk = accs[k] * tk
                    pk = tk * ck
                    npart = nk if npart is None else npart + nk
                    ppart = pk if ppart is None else ppart + pk
                op[p][b, :] = ppart
                on[p][b, :] = npart
                return carry

            lax.fori_loop(0, SB, bbody, 0)
            base = wid * CHUNK + s * SB
            pltpu.async_copy(op[p], pos_out.at[pl.ds(base, SB), :], osems[p])
            pltpu.async_copy(on[p], neg_out.at[pl.ds(base, SB), :], osems[p])

        issue(0, 0, True)
        issue(1, 1, False)
        drain(0)
        compute(0, 0)
        issue(2, 0, True)
        drain(1)
        compute(1, 1)
        issue(3, 1, False)

        def lbody(i, carry):
            for p in range(2):
                cur = 2 * i + 2 + p
                drain(p)
                odrain(p)
                compute(cur, p)
                issue(cur + 2, p, p == 0)
            return carry

        lax.fori_loop(0, STEPS // 2 - 2, lbody, 0)
        for p in range(2):
            drain(p)
            odrain(p)
            compute(STEPS - 2 + p, p)
        odrain(0)
        odrain(1)

    return sc_kernel


def _tc_reduce_body(p_ref, n_ref, o_ref):
    ri = lax.broadcasted_iota(jnp.int32, (128, 128 // LANES), 0)
    cj = lax.broadcasted_iota(jnp.int32, (128, 128 // LANES), 1)
    sel = jnp.where(ri // LANES == cj, 1.0, 0.0).astype(jnp.float32)
    pos = lax.dot(p_ref[...], sel, precision=lax.Precision.HIGHEST)
    neg = lax.dot(n_ref[...], sel, precision=lax.Precision.HIGHEST)

    def logsig(x):
        return jnp.minimum(x, 0.0) - jnp.log1p(jnp.exp(-jnp.abs(x)))

    o_ref[0, 0] = -(jnp.sum(logsig(pos)) + jnp.sum(logsig(-neg)))


def kernel(target_word, context_word, negative_example, W_target, W_context):
    B = target_word.shape[0]
    NEG = negative_example.shape[1]
    V = W_target.shape[0]

    t2 = target_word.astype(jnp.int32).reshape(-1, 16)
    c2 = context_word.astype(jnp.int32).reshape(-1, 16)
    n2 = negative_example.astype(jnp.int32).reshape(B * NEG // 128, 128)

    w2 = _tc_fuse(W_target, W_context, V)
    pos_p, neg_p = _sc_partials(B, NEG, V)(t2, c2, n2, w2)

    reduce_call = pl.pallas_call(
        _tc_reduce_body,
        out_shape=jax.ShapeDtypeStruct((1, 1), jnp.float32),
        out_specs=pl.BlockSpec(memory_space=pltpu.SMEM),
    )
    loss = reduce_call(pos_p.reshape(-1, 128), neg_p.reshape(-1, 128))
    return loss[0, 0]


# speedup vs baseline: 1.0009x; 1.0008x over previous
import functools

import jax
import jax.numpy as jnp
from jax import lax
from jax.experimental import pallas as pl
from jax.experimental.pallas import tpu as pltpu
from jax.experimental.pallas import tpu_sc as plsc

EMB = 64
ROW = 2 * EMB
LANES = 16
VPR = EMB // LANES

_info = plsc.get_sparse_core_info()
NC, NS = _info.num_cores, _info.num_subcores
NW = NC * NS


def _tc_fuse_body(wt_ref, wc_ref, o_ref):
    o_ref[:, :EMB] = wt_ref[...]
    o_ref[:, EMB:] = wc_ref[...]


def _tc_fuse(wt, wc, V):
    C = 20000
    return pl.pallas_call(
        _tc_fuse_body,
        grid=(V // C,),
        in_specs=[
            pl.BlockSpec((C, EMB), lambda i: (i, 0)),
            pl.BlockSpec((C, EMB), lambda i: (i, 0)),
        ],
        out_specs=pl.BlockSpec((C, ROW), lambda i: (i, 0)),
        out_shape=jax.ShapeDtypeStruct((V, ROW), jnp.float32),
    )(wt, wc)


def _sc_partials(B, NEG, V):
    CHUNK = B // NW
    SB = 16
    STEPS = CHUNK // SB
    RPS = SB * NEG
    NROWS = CHUNK * NEG // 128
    assert CHUNK % SB == 0 and B % NW == 0 and RPS == 320

    mesh = plsc.VectorSubcoreMesh(core_axis_name="c", subcore_axis_name="s")

    @functools.partial(
        pl.kernel,
        mesh=mesh,
        compiler_params=pltpu.CompilerParams(use_tc_tiling_on_sc=False),
        out_type=[
            jax.ShapeDtypeStruct((B, LANES), jnp.float32),
            jax.ShapeDtypeStruct((B, LANES), jnp.float32),
        ],
        scratch_types=[
            pltpu.VMEM((STEPS, SB), jnp.int32),
            pltpu.VMEM((STEPS, SB), jnp.int32),
            pltpu.VMEM((NROWS, 128), jnp.int32),
            pltpu.VMEM((SB, ROW), jnp.float32),
            pltpu.VMEM((SB, ROW), jnp.float32),
            pltpu.VMEM((SB, ROW), jnp.float32),
            pltpu.VMEM((SB, ROW), jnp.float32),
            pltpu.VMEM((RPS, ROW), jnp.float32),
            pltpu.VMEM((RPS, ROW), jnp.float32),
            pltpu.VMEM((SB, LANES), jnp.float32),
            pltpu.VMEM((SB, LANES), jnp.float32),
            pltpu.VMEM((SB, LANES), jnp.float32),
            pltpu.VMEM((SB, LANES), jnp.float32),
            pltpu.SemaphoreType.DMA,
            pltpu.SemaphoreType.DMA,
            pltpu.SemaphoreType.DMA,
            pltpu.SemaphoreType.DMA,
        ],
    )
    def sc_kernel(tgt_idx_hbm, ctx_idx_hbm, neg_idx_hbm, w2_hbm,
                  pos_out, neg_out,
                  tgt_idx_v, ctx_idx_v, neg_idx_v,
                  tb0, tb1, cb0, cb1, rb0, rb1,
                  op0, op1, on0, on1, sem0, sem1, osem0, osem1):
        wid = lax.axis_index("s") * NC + lax.axis_index("c")
        tb = (tb0, tb1)
        cb = (cb0, cb1)
        rb = (rb0, rb1)
        op = (op0, op1)
        on = (on0, on1)
        sems = (sem0, sem1)
        osems = (osem0, osem1)

        pltpu.sync_copy(tgt_idx_hbm.at[pl.ds(wid * STEPS, STEPS), :], tgt_idx_v)
        pltpu.sync_copy(ctx_idx_hbm.at[pl.ds(wid * STEPS, STEPS), :], ctx_idx_v)
        pltpu.sync_copy(neg_idx_hbm.at[pl.ds(wid * NROWS, NROWS), :], neg_idx_v)

        def issue(s, p, even):
            j = s // 2
            if even:
                parts = ((5 * j, 0, 128, 0), (5 * j + 1, 0, 128, 128),
                         (5 * j + 2, 0, 64, 256))
            else:
                parts = ((5 * j + 2, 64, 64, 0), (5 * j + 3, 0, 128, 64),
                         (5 * j + 4, 0, 128, 192))
            for row, off, n, dst in parts:
                pltpu.async_copy(
                    w2_hbm.at[neg_idx_v.at[row, pl.ds(off, n)]],
                    rb[p].at[pl.ds(dst, n), :],
                    sems[p])
            pltpu.async_copy(w2_hbm.at[tgt_idx_v.at[s]], tb[p], sems[p])
            pltpu.async_copy(w2_hbm.at[ctx_idx_v.at[s]], cb[p], sems[p])

        def drain(p):
            pltpu.make_async_copy(w2_hbm.at[pl.ds(0, RPS), :], rb[p], sems[p]).wait()
            pltpu.make_async_copy(w2_hbm.at[pl.ds(0, SB), :], tb[p], sems[p]).wait()
            pltpu.make_async_copy(w2_hbm.at[pl.ds(0, SB), :], cb[p], sems[p]).wait()

        def odrain(p):
            pltpu.make_async_copy(pos_out.at[pl.ds(0, SB), :], op[p], osems[p]).wait()
            pltpu.make_async_copy(pos_out.at[pl.ds(0, SB), :], on[p], osems[p]).wait()

        def compute(s, p):
            rbp, tbp, cbp = rb[p], tb[p], cb[p]

            def bbody(b, carry):
                r0 = b * NEG
                accs = [rbp[r0, pl.ds(EMB + k * LANES, LANES)]
                        for k in range(VPR)]
                for j in range(1, NEG):
                    for k in range(VPR):
                        accs[k] = accs[k] + rbp[r0 + j,
                                                pl.ds(EMB + k * LANES, LANES)]
                npart = None
                ppart = None
                for k in range(VPR):
                    tk = tbp[b, pl.ds(k * LANES, LANES)]
                    ck = cbp[b, pl.ds(EMB + k * LANES, LANES)]
                    nk = accs[k] * tk
                    pk = tk * ck
                    npart = nk if npart is None else npart + nk
                    ppart = pk if ppart is None else ppart + pk
                op[p][b, :] = ppart
                on[p][b, :] = npart
                return carry

            lax.fori_loop(0, SB, bbody, 0)
            base = wid * CHUNK + s * SB
            pltpu.async_copy(op[p], pos_out.at[pl.ds(base, SB), :], osems[p])
            pltpu.async_copy(on[p], neg_out.at[pl.ds(base, SB), :], osems[p])

        issue(0, 0, True)
        issue(1, 1, False)
        drain(0)
        compute(0, 0)
        issue(2, 0, True)
        drain(1)
        compute(1, 1)
        issue(3, 1, False)

        def lbody(i, carry):
            for p in range(2):
                cur = 2 * i + 2 + p
                drain(p)
                odrain(p)
                compute(cur, p)
                issue(cur + 2, p, p == 0)
            return carry

        lax.fori_loop(0, STEPS // 2 - 2, lbody, 0)
        for p in range(2):
            drain(p)
            odrain(p)
            compute(STEPS - 2 + p, p)
        odrain(0)
        odrain(1)

    return sc_kernel


def _tc_reduce_body(p_ref, n_ref, o_ref):
    ri = lax.broadcasted_iota(jnp.int32, (128, 128 // LANES), 0)
    cj = lax.broadcasted_iota(jnp.int32, (128, 128 // LANES), 1)
    sel = jnp.where(ri // LANES == cj, 1.0, 0.0).astype(jnp.float32)
    pos = lax.dot(p_ref[...], sel, precision=lax.Precision.HIGHEST)
    neg = lax.dot(n_ref[...], sel, precision=lax.Precision.HIGHEST)

    def logsig(x):
        return jnp.minimum(x, 0.0) - jnp.log1p(jnp.exp(-jnp.abs(x)))

    o_ref[0, 0] = -(jnp.sum(logsig(pos)) + jnp.sum(logsig(-neg)))


def kernel(target_word, context_word, negative_example, W_target, W_context):
    B = target_word.shape[0]
    NEG = negative_example.shape[1]
    V = W_target.shape[0]

    t2 = target_word.astype(jnp.int32).reshape(-1, 16)
    c2 = context_word.astype(jnp.int32).reshape(-1, 16)
    n2 = negative_example.astype(jnp.int32).reshape(B * NEG // 128, 128)

    w2 = _tc_fuse(W_target, W_context, V)
    pos_p, neg_p = _sc_partials(B, NEG, V)(t2, c2, n2, w2)

    reduce_call = pl.pallas_call(
        _tc_reduce_body,
        out_shape=jax.ShapeDtypeStruct((1, 1), jnp.float32),
        out_specs=pl.BlockSpec(memory_space=pltpu.SMEM),
    )
    loss = reduce_call(pos_p.reshape(-1, 128), neg_p.reshape(-1, 128))
    return loss[0, 0]


# speedup vs baseline: 1.0917x; 1.0908x over previous
import functools

import jax
import jax.numpy as jnp
from jax import lax
from jax.experimental import pallas as pl
from jax.experimental.pallas import tpu as pltpu
from jax.experimental.pallas import tpu_sc as plsc

EMB = 64
LANES = 16
VPR = EMB // LANES

_info = plsc.get_sparse_core_info()
NC, NS = _info.num_cores, _info.num_subcores
NW = NC * NS


def _sc_partials(B, NEG, V):
    CHUNK = B // NW
    S = 32
    STEPS = CHUNK // S
    NEG_CH = (S * NEG) // 128
    NROWS = CHUNK * NEG // 128
    assert S * NEG % 128 == 0 and CHUNK % S == 0 and B % NW == 0

    mesh = plsc.VectorSubcoreMesh(core_axis_name="c", subcore_axis_name="s")

    @functools.partial(
        pl.kernel,
        mesh=mesh,
        compiler_params=pltpu.CompilerParams(use_tc_tiling_on_sc=False),
        out_type=[
            jax.ShapeDtypeStruct((B, LANES), jnp.float32),
            jax.ShapeDtypeStruct((B, LANES), jnp.float32),
        ],
        scratch_types=[
            pltpu.VMEM((STEPS, S), jnp.int32),
            pltpu.VMEM((STEPS, S), jnp.int32),
            pltpu.VMEM((NROWS, 128), jnp.int32),
            pltpu.VMEM((S, EMB), jnp.float32),
            pltpu.VMEM((S, EMB), jnp.float32),
            pltpu.VMEM((S, EMB), jnp.float32),
            pltpu.VMEM((S, EMB), jnp.float32),
            pltpu.VMEM((S * NEG, EMB), jnp.float32),
            pltpu.VMEM((S * NEG, EMB), jnp.float32),
            pltpu.VMEM((CHUNK, LANES), jnp.float32),
            pltpu.VMEM((CHUNK, LANES), jnp.float32),
            pltpu.SemaphoreType.DMA,
            pltpu.SemaphoreType.DMA,
        ],
    )
    def sc_kernel(tgt_idx_hbm, ctx_idx_hbm, neg_idx_hbm, wt_hbm, wc_hbm,
                  pos_out, neg_out,
                  tgt_idx_v, ctx_idx_v, neg_idx_v,
                  tb0, tb1, cb0, cb1, rb0, rb1,
                  outp_v, outn_v, sem0, sem1):
        wid = lax.axis_index("s") * NC + lax.axis_index("c")
        tb = (tb0, tb1)
        cb = (cb0, cb1)
        rb = (rb0, rb1)
        sems = (sem0, sem1)

        pltpu.sync_copy(tgt_idx_hbm.at[pl.ds(wid * STEPS, STEPS), :], tgt_idx_v)
        pltpu.sync_copy(ctx_idx_hbm.at[pl.ds(wid * STEPS, STEPS), :], ctx_idx_v)
        pltpu.sync_copy(neg_idx_hbm.at[pl.ds(wid * NROWS, NROWS), :], neg_idx_v)

        def issue(s, p):
            for ch in range(NEG_CH):
                pltpu.async_copy(
                    wc_hbm.at[neg_idx_v.at[s * NEG_CH + ch]],
                    rb[p].at[pl.ds(ch * 128, 128), :],
                    sems[p])
            pltpu.async_copy(wt_hbm.at[tgt_idx_v.at[s]], tb[p], sems[p])
            pltpu.async_copy(wc_hbm.at[ctx_idx_v.at[s]], cb[p], sems[p])

        def drain(p):
            pltpu.make_async_copy(wc_hbm.at[pl.ds(0, S * NEG), :], rb[p], sems[p]).wait()
            pltpu.make_async_copy(wt_hbm.at[pl.ds(0, S), :], tb[p], sems[p]).wait()
            pltpu.make_async_copy(wc_hbm.at[pl.ds(0, S), :], cb[p], sems[p]).wait()

        def compute(s, p):
            rbp, tbp, cbp = rb[p], tb[p], cb[p]

            def bbody(b, carry):
                g = s * S + b
                r0 = b * NEG
                accs = [rbp[r0, pl.ds(k * LANES, LANES)] for k in range(VPR)]
                for j in range(1, NEG):
                    for k in range(VPR):
                        accs[k] = accs[k] + rbp[r0 + j, pl.ds(k * LANES, LANES)]
                npart = None
                ppart = None
                for k in range(VPR):
                    tk = tbp[b, pl.ds(k * LANES, LANES)]
                    ck = cbp[b, pl.ds(k * LANES, LANES)]
                    nk = accs[k] * tk
                    pk = tk * ck
                    npart = nk if npart is None else npart + nk
                    ppart = pk if ppart is None else ppart + pk
                outp_v[g, :] = ppart
                outn_v[g, :] = npart
                return carry

            lax.fori_loop(0, S, bbody, 0)

        issue(0, 0)
        issue(1, 1)

        def lbody(i, carry):
            for p in range(2):
                cur = 2 * i + p
                drain(p)
                compute(cur, p)
                issue(cur + 2, p)
            return carry

        lax.fori_loop(0, STEPS // 2 - 1, lbody, 0)
        for p in range(2):
            drain(p)
            compute(STEPS - 2 + p, p)

        pltpu.sync_copy(outp_v, pos_out.at[pl.ds(wid * CHUNK, CHUNK), :])
        pltpu.sync_copy(outn_v, neg_out.at[pl.ds(wid * CHUNK, CHUNK), :])

    return sc_kernel


def _tc_reduce_body(p_ref, n_ref, o_ref):
    ri = lax.broadcasted_iota(jnp.int32, (128, 128 // LANES), 0)
    cj = lax.broadcasted_iota(jnp.int32, (128, 128 // LANES), 1)
    sel = jnp.where(ri // LANES == cj, 1.0, 0.0).astype(jnp.float32)
    pos = lax.dot(p_ref[...], sel, precision=lax.Precision.HIGHEST)
    neg = lax.dot(n_ref[...], sel, precision=lax.Precision.HIGHEST)

    def logsig(x):
        return jnp.minimum(x, 0.0) - jnp.log1p(jnp.exp(-jnp.abs(x)))

    o_ref[0, 0] = -(jnp.sum(logsig(pos)) + jnp.sum(logsig(-neg)))


def kernel(target_word, context_word, negative_example, W_target, W_context):
    B = target_word.shape[0]
    NEG = negative_example.shape[1]
    V = W_target.shape[0]

    t2 = target_word.astype(jnp.int32).reshape(B // 32, 32)
    c2 = context_word.astype(jnp.int32).reshape(B // 32, 32)
    n2 = negative_example.astype(jnp.int32).reshape(B * NEG // 128, 128)

    pos_p, neg_p = _sc_partials(B, NEG, V)(t2, c2, n2, W_target, W_context)

    reduce_call = pl.pallas_call(
        _tc_reduce_body,
        out_shape=jax.ShapeDtypeStruct((1, 1), jnp.float32),
        out_specs=pl.BlockSpec(memory_space=pltpu.SMEM),
    )
    loss = reduce_call(pos_p.reshape(-1, 128), neg_p.reshape(-1, 128))
    return loss[0, 0]
